# baseline (speedup 1.0000x reference)
import functools

import jax
import jax.numpy as jnp
from jax import lax
from jax.experimental import pallas as pl
from jax.experimental.pallas import tpu as pltpu
from jax.experimental.pallas import tpu_sc as plsc

N = 10000
E = 320000
D = 128
DH = D // 2
DE = 16
L = 3

NC = 2
NS = 16
EPT = E // NS
C = 80
NCHUNK = EPT // C
ROWS_PT = 632
NPAD = NS * ROWS_PT

_SC_MESH = plsc.VectorSubcoreMesh(
    core_axis_name="c", subcore_axis_name="s", num_cores=NC, num_subcores=NS
)


NOUTER = NCHUNK // 2
ZR = 104


@functools.partial(
    pl.kernel,
    out_type=pltpu.HBM((NC, NPAD, DH), jnp.float32),
    mesh=_SC_MESH,
    scratch_types=[
        pltpu.VMEM((NCHUNK, C), jnp.int32),
        pltpu.VMEM((NCHUNK, C), jnp.int32),
        [pltpu.VMEM((C, DH), jnp.float32)] * 2,
        [pltpu.VMEM((C, DH), jnp.float32)] * 2,
        [pltpu.VMEM((C, DH), jnp.float32)] * 2,
        pltpu.VMEM((ZR, DH), jnp.float32),
        pltpu.VMEM_SHARED((NPAD, DH), jnp.float32),
        [pltpu.SemaphoreType.DMA] * 2,
        [pltpu.SemaphoreType.DMA] * 2,
        [pltpu.SemaphoreType.DMA] * 2,
    ],
    compiler_params=pltpu.CompilerParams(use_tc_tiling_on_sc=False),
)
def _gather_scatter(h_hbm, ee_hbm, srci_hbm, dsti_hbm, out_hbm,
                    srcall, dstall, hb, eb, sb, zero_v, acc,
                    sem_h, sem_e, sem_s):
    c = lax.axis_index("c")
    s = lax.axis_index("s")
    base0 = s * EPT

    pltpu.sync_copy(srci_hbm.at[s], srcall)
    pltpu.sync_copy(dsti_hbm.at[s], dstall)

    for b in range(2):
        pltpu.async_copy(ee_hbm.at[pl.ds(base0 + b * C, C), c], eb[b],
                         sem_e[b])
        pltpu.async_copy(h_hbm.at[c].at[srcall.at[b]], hb[b], sem_h[b])

    def zbody(i, carry):
        for j in range(DH // 16):
            zero_v[i, pl.ds(j * 16, 16)] = jnp.zeros((16,), jnp.float32)
        return carry

    lax.fori_loop(0, ZR, zbody, None)
    r0 = s * ROWS_PT
    for k in range(ROWS_PT // ZR):
        pltpu.sync_copy(zero_v, acc.at[pl.ds(r0 + k * ZR, ZR)])
    pltpu.sync_copy(zero_v.at[pl.ds(0, ROWS_PT % ZR)],
                    acc.at[pl.ds(r0 + (ROWS_PT // ZR) * ZR, ROWS_PT % ZR)])
    plsc.subcore_barrier()

    def outer(g, carry):
        for b in range(2):
            i = 2 * g + b
            pltpu.make_async_copy(ee_hbm.at[pl.ds(base0 + i * C, C), c],
                                  eb[b], sem_e[b]).wait()
            pltpu.make_async_copy(h_hbm.at[c].at[srcall.at[i]], hb[b],
                                  sem_h[b]).wait()

            @pl.when(g > 0)
            def _():
                pltpu.make_async_copy(sb[b], acc.at[dstall.at[i - 2]],
                                      sem_s[b]).wait()

            def relu_row(r, inner):
                for j in range(DH // 16):
                    sl = pl.ds(j * 16, 16)
                    sb[b][r, sl] = jnp.maximum(hb[b][r, sl] + eb[b][r, sl],
                                               0.0)
                return inner

            lax.fori_loop(0, C, relu_row, None)

            @pl.when(g < NOUTER - 1)
            def _():
                nxt = i + 2
                pltpu.async_copy(ee_hbm.at[pl.ds(base0 + nxt * C, C), c],
                                 eb[b], sem_e[b])
                pltpu.async_copy(h_hbm.at[c].at[srcall.at[nxt]], hb[b],
                                 sem_h[b])

            pltpu.async_copy(sb[b], acc.at[dstall.at[i]], sem_s[b],
                             add=True)
        return carry

    lax.fori_loop(0, NOUTER, outer, None)

    for b in range(2):
        pltpu.make_async_copy(sb[b], acc.at[dstall.at[NCHUNK - 2 + b]],
                              sem_s[b]).wait()

    plsc.subcore_barrier()
    pltpu.sync_copy(acc.at[pl.ds(r0, ROWS_PT)],
                    out_hbm.at[c, pl.ds(r0, ROWS_PT)])


def _enc_body(x_ref, w_ref, b_ref, o_ref, oh_ref):
    h = (
        jnp.dot(x_ref[...], w_ref[...], preferred_element_type=jnp.float32)
        + b_ref[...]
    )
    o_ref[...] = h
    oh_ref[0] = h[:, :DH]
    oh_ref[1] = h[:, DH:]


def _encoder(x, W0, b0):
    return pl.pallas_call(
        _enc_body,
        out_shape=[
            jax.ShapeDtypeStruct((N, D), jnp.float32),
            jax.ShapeDtypeStruct((NC, N, DH), jnp.float32),
        ],
    )(x, W0, b0.reshape(1, D))


_EB = 3200


def _ee_body(ea_ref, w_ref, b_ref, o0_ref, o1_ref, o2_ref):
    p = (
        jnp.dot(ea_ref[...], w_ref[...], preferred_element_type=jnp.float32)
        + b_ref[...]
    )
    for l, o_ref in enumerate((o0_ref, o1_ref, o2_ref)):
        o_ref[...] = p[:, l * D:(l + 1) * D]


def _ee_all(edge_attr, We, be):
    Wc = jnp.transpose(We, (1, 0, 2)).reshape(DE, L * D)
    bc = be.reshape(1, L * D)
    return pl.pallas_call(
        _ee_body,
        grid=(E // _EB,),
        in_specs=[
            pl.BlockSpec((_EB, DE), lambda i: (i, 0)),
            pl.BlockSpec((DE, L * D), lambda i: (0, 0)),
            pl.BlockSpec((1, L * D), lambda i: (0, 0)),
        ],
        out_specs=[pl.BlockSpec((_EB, D), lambda i: (i, 0))] * 3,
        out_shape=[jax.ShapeDtypeStruct((E, D), jnp.float32)] * 3,
    )(edge_attr, Wc, bc)


def _dense_body(relu_out, h_ref, p_ref, eps_ref, w1_ref, b1_ref, g1_ref,
                bb1_ref, w2_ref, b2_ref, go_ref, bo_ref, o_ref, oh_ref):
    aggr = jnp.concatenate([p_ref[0, :N], p_ref[1, :N]], axis=1)
    u = (1.0 + eps_ref[0]) * h_ref[...] + aggr
    t = (
        jnp.dot(u, w1_ref[...], preferred_element_type=jnp.float32)
        + b1_ref[...]
    )
    m = jnp.mean(t, axis=0, keepdims=True)
    v = jnp.mean(jnp.square(t - m), axis=0, keepdims=True)
    t = (t - m) * lax.rsqrt(v + 1e-5) * g1_ref[...] + bb1_ref[...]
    t = jnp.maximum(t, 0.0)
    o = (
        jnp.dot(t, w2_ref[...], preferred_element_type=jnp.float32)
        + b2_ref[...]
    )
    m2 = jnp.mean(o, axis=0, keepdims=True)
    v2 = jnp.mean(jnp.square(o - m2), axis=0, keepdims=True)
    o = (o - m2) * lax.rsqrt(v2 + 1e-5) * go_ref[...] + bo_ref[...]
    if relu_out:
        o = jnp.maximum(o, 0.0)
    o_ref[...] = o
    oh_ref[0] = o[:, :DH]
    oh_ref[1] = o[:, DH:]


def _dense(relu_out, h, p, eps_l, W1l, b1l, g1l, bb1l, W2l, b2l, gOl, bOl):
    vspec = pl.BlockSpec(memory_space=pltpu.MemorySpace.VMEM)
    sspec = pl.BlockSpec(memory_space=pltpu.SMEM)
    return pl.pallas_call(
        functools.partial(_dense_body, relu_out),
        in_specs=[vspec, vspec, sspec] + [vspec] * 8,
        out_specs=[vspec, vspec],
        out_shape=[
            jax.ShapeDtypeStruct((N, D), jnp.float32),
            jax.ShapeDtypeStruct((NC, N, DH), jnp.float32),
        ],
    )(h, p, eps_l.reshape(1), W1l, b1l.reshape(1, 2 * D),
      g1l.reshape(1, 2 * D), bb1l.reshape(1, 2 * D), W2l,
      b2l.reshape(1, D), gOl.reshape(1, D), bOl.reshape(1, D))


def kernel(x, edge_index, edge_attr, batch, W0, b0, We, be, W1, b1, g1, bb1,
           W2, b2, gO, bO, eps):
    src = edge_index[0].reshape(NS, NCHUNK, C)
    dsti = edge_index[1].reshape(NS, NCHUNK, C)
    h, hh = _encoder(x, W0, b0)
    ee = _ee_all(edge_attr, We, be)
    for l in range(L):
        partials = _gather_scatter(hh, ee[l].reshape(E, NC, DH), src, dsti)
        h, hh = _dense(l < L - 1, h, partials, eps[l], W1[l], b1[l], g1[l],
                       bb1[l], W2[l], b2[l], gO[l], bO[l])
    return h


# speedup vs baseline: 1.9572x; 1.9572x over previous
import functools

import jax
import jax.numpy as jnp
from jax import lax
from jax.experimental import pallas as pl
from jax.experimental.pallas import tpu as pltpu
from jax.experimental.pallas import tpu_sc as plsc

N = 10000
E = 320000
D = 128
DH = D // 2
DE = 16
L = 3

NC = 2
NS = 16
EPT = E // NS
C = 80
NCHUNK = EPT // C
ROWS_PT = 632
NPAD = NS * ROWS_PT

_SC_MESH = plsc.VectorSubcoreMesh(
    core_axis_name="c", subcore_axis_name="s", num_cores=NC, num_subcores=NS
)


NOUTER = NCHUNK // 2
ZR = 104


@functools.partial(
    pl.kernel,
    out_type=pltpu.HBM((NC, NPAD, DH), jnp.float32),
    mesh=_SC_MESH,
    scratch_types=[
        pltpu.VMEM((NCHUNK, C), jnp.int32),
        pltpu.VMEM((NCHUNK, C), jnp.int32),
        [pltpu.VMEM((C, DH), jnp.float32)] * 2,
        [pltpu.VMEM((C, D), jnp.float32)] * 2,
        [pltpu.VMEM((C, DH), jnp.float32)] * 2,
        pltpu.VMEM((ZR, DH), jnp.float32),
        pltpu.VMEM_SHARED((NPAD, DH), jnp.float32),
        [pltpu.SemaphoreType.DMA] * 2,
        [pltpu.SemaphoreType.DMA] * 2,
        [pltpu.SemaphoreType.DMA] * 2,
    ],
    compiler_params=pltpu.CompilerParams(use_tc_tiling_on_sc=False),
)
def _gather_scatter(h_hbm, ee_hbm, srci_hbm, dsti_hbm, out_hbm,
                    srcall, dstall, hb, eb, sb, zero_v, acc,
                    sem_h, sem_e, sem_s):
    c = lax.axis_index("c")
    s = lax.axis_index("s")
    base0 = s * EPT

    pltpu.sync_copy(srci_hbm.at[s], srcall)
    pltpu.sync_copy(dsti_hbm.at[s], dstall)

    for b in range(2):
        pltpu.async_copy(ee_hbm.at[pl.ds(base0 + b * C, C)], eb[b],
                         sem_e[b])
        pltpu.async_copy(h_hbm.at[c].at[srcall.at[b]], hb[b], sem_h[b])

    def zbody(i, carry):
        for j in range(DH // 16):
            zero_v[i, pl.ds(j * 16, 16)] = jnp.zeros((16,), jnp.float32)
        return carry

    lax.fori_loop(0, ZR, zbody, None)
    r0 = s * ROWS_PT
    for k in range(ROWS_PT // ZR):
        pltpu.sync_copy(zero_v, acc.at[pl.ds(r0 + k * ZR, ZR)])
    pltpu.sync_copy(zero_v.at[pl.ds(0, ROWS_PT % ZR)],
                    acc.at[pl.ds(r0 + (ROWS_PT // ZR) * ZR, ROWS_PT % ZR)])
    plsc.subcore_barrier()

    def outer(g, carry):
        for b in range(2):
            i = 2 * g + b
            pltpu.make_async_copy(ee_hbm.at[pl.ds(base0 + i * C, C)],
                                  eb[b], sem_e[b]).wait()
            pltpu.make_async_copy(h_hbm.at[c].at[srcall.at[i]], hb[b],
                                  sem_h[b]).wait()

            @pl.when(g > 0)
            def _():
                pltpu.make_async_copy(sb[b], acc.at[dstall.at[i - 2]],
                                      sem_s[b]).wait()

            coff = c * DH

            def relu_row(r, inner):
                for j in range(DH // 16):
                    sl = pl.ds(j * 16, 16)
                    esl = pl.ds(coff + j * 16, 16)
                    sb[b][r, sl] = jnp.maximum(hb[b][r, sl] + eb[b][r, esl],
                                               0.0)
                return inner

            lax.fori_loop(0, C, relu_row, None)

            @pl.when(g < NOUTER - 1)
            def _():
                nxt = i + 2
                pltpu.async_copy(ee_hbm.at[pl.ds(base0 + nxt * C, C)],
                                 eb[b], sem_e[b])
                pltpu.async_copy(h_hbm.at[c].at[srcall.at[nxt]], hb[b],
                                 sem_h[b])

            pltpu.async_copy(sb[b], acc.at[dstall.at[i]], sem_s[b],
                             add=True)
        return carry

    lax.fori_loop(0, NOUTER, outer, None)

    for b in range(2):
        pltpu.make_async_copy(sb[b], acc.at[dstall.at[NCHUNK - 2 + b]],
                              sem_s[b]).wait()

    plsc.subcore_barrier()
    pltpu.sync_copy(acc.at[pl.ds(r0, ROWS_PT)],
                    out_hbm.at[c, pl.ds(r0, ROWS_PT)])


def _enc_body(x_ref, w_ref, b_ref, o_ref, oh_ref):
    h = (
        jnp.dot(x_ref[...], w_ref[...], preferred_element_type=jnp.float32)
        + b_ref[...]
    )
    o_ref[...] = h
    oh_ref[0] = h[:, :DH]
    oh_ref[1] = h[:, DH:]


def _encoder(x, W0, b0):
    return pl.pallas_call(
        _enc_body,
        out_shape=[
            jax.ShapeDtypeStruct((N, D), jnp.float32),
            jax.ShapeDtypeStruct((NC, N, DH), jnp.float32),
        ],
    )(x, W0, b0.reshape(1, D))


_EB = 3200


def _ee_body(ea_ref, w_ref, b_ref, o0_ref, o1_ref, o2_ref):
    p = (
        jnp.dot(ea_ref[...], w_ref[...], preferred_element_type=jnp.float32)
        + b_ref[...]
    )
    for l, o_ref in enumerate((o0_ref, o1_ref, o2_ref)):
        o_ref[...] = p[:, l * D:(l + 1) * D]


def _ee_all(edge_attr, We, be):
    Wc = jnp.transpose(We, (1, 0, 2)).reshape(DE, L * D)
    bc = be.reshape(1, L * D)
    return pl.pallas_call(
        _ee_body,
        grid=(E // _EB,),
        in_specs=[
            pl.BlockSpec((_EB, DE), lambda i: (i, 0)),
            pl.BlockSpec((DE, L * D), lambda i: (0, 0)),
            pl.BlockSpec((1, L * D), lambda i: (0, 0)),
        ],
        out_specs=[pl.BlockSpec((_EB, D), lambda i: (i, 0))] * 3,
        out_shape=[jax.ShapeDtypeStruct((E, D), jnp.float32)] * 3,
    )(edge_attr, Wc, bc)


def _dense_body(relu_out, h_ref, p_ref, eps_ref, w1_ref, b1_ref, g1_ref,
                bb1_ref, w2_ref, b2_ref, go_ref, bo_ref, o_ref, oh_ref):
    aggr = jnp.concatenate([p_ref[0, :N], p_ref[1, :N]], axis=1)
    u = (1.0 + eps_ref[0]) * h_ref[...] + aggr
    t = (
        jnp.dot(u, w1_ref[...], preferred_element_type=jnp.float32)
        + b1_ref[...]
    )
    m = jnp.mean(t, axis=0, keepdims=True)
    v = jnp.mean(jnp.square(t - m), axis=0, keepdims=True)
    t = (t - m) * lax.rsqrt(v + 1e-5) * g1_ref[...] + bb1_ref[...]
    t = jnp.maximum(t, 0.0)
    o = (
        jnp.dot(t, w2_ref[...], preferred_element_type=jnp.float32)
        + b2_ref[...]
    )
    m2 = jnp.mean(o, axis=0, keepdims=True)
    v2 = jnp.mean(jnp.square(o - m2), axis=0, keepdims=True)
    o = (o - m2) * lax.rsqrt(v2 + 1e-5) * go_ref[...] + bo_ref[...]
    if relu_out:
        o = jnp.maximum(o, 0.0)
    o_ref[...] = o
    oh_ref[0] = o[:, :DH]
    oh_ref[1] = o[:, DH:]


def _dense(relu_out, h, p, eps_l, W1l, b1l, g1l, bb1l, W2l, b2l, gOl, bOl):
    vspec = pl.BlockSpec(memory_space=pltpu.MemorySpace.VMEM)
    sspec = pl.BlockSpec(memory_space=pltpu.SMEM)
    return pl.pallas_call(
        functools.partial(_dense_body, relu_out),
        in_specs=[vspec, vspec, sspec] + [vspec] * 8,
        out_specs=[vspec, vspec],
        out_shape=[
            jax.ShapeDtypeStruct((N, D), jnp.float32),
            jax.ShapeDtypeStruct((NC, N, DH), jnp.float32),
        ],
    )(h, p, eps_l.reshape(1), W1l, b1l.reshape(1, 2 * D),
      g1l.reshape(1, 2 * D), bb1l.reshape(1, 2 * D), W2l,
      b2l.reshape(1, D), gOl.reshape(1, D), bOl.reshape(1, D))


def kernel(x, edge_index, edge_attr, batch, W0, b0, We, be, W1, b1, g1, bb1,
           W2, b2, gO, bO, eps):
    src = edge_index[0].reshape(NS, NCHUNK, C)
    dsti = edge_index[1].reshape(NS, NCHUNK, C)
    h, hh = _encoder(x, W0, b0)
    ee = _ee_all(edge_attr, We, be)
    for l in range(L):
        partials = _gather_scatter(hh, ee[l], src, dsti)
        h, hh = _dense(l < L - 1, h, partials, eps[l], W1[l], b1[l], g1[l],
                       bb1[l], W2[l], b2[l], gO[l], bO[l])
    return h


# speedup vs baseline: 3.2093x; 1.6397x over previous
import functools

import jax
import jax.numpy as jnp
from jax import lax
from jax.experimental import pallas as pl
from jax.experimental.pallas import tpu as pltpu
from jax.experimental.pallas import tpu_sc as plsc

N = 10000
E = 320000
D = 128
DH = D // 2
DE = 16
L = 3

NC = 2
NS = 16
EPT = E // NS
C = 80
NCHUNK = EPT // C
ROWS_PT = 632
NPAD = NS * ROWS_PT

_SC_MESH = plsc.VectorSubcoreMesh(
    core_axis_name="c", subcore_axis_name="s", num_cores=NC, num_subcores=NS
)


NOUTER = NCHUNK // 2
ZR = 104


@functools.partial(
    pl.kernel,
    out_type=pltpu.HBM((NC, NPAD, DH), jnp.float32),
    mesh=_SC_MESH,
    scratch_types=[
        pltpu.VMEM((NCHUNK, C), jnp.int32),
        pltpu.VMEM((NCHUNK, C), jnp.int32),
        [pltpu.VMEM((C, DH), jnp.float32)] * 2,
        [pltpu.VMEM((C // 2, D), jnp.float32)] * 2,
        [pltpu.VMEM((C, DH), jnp.float32)] * 2,
        pltpu.VMEM((ZR, DH), jnp.float32),
        pltpu.VMEM_SHARED((NPAD, DH), jnp.float32),
        [pltpu.SemaphoreType.DMA] * 2,
        [pltpu.SemaphoreType.DMA] * 2,
        [pltpu.SemaphoreType.DMA] * 2,
    ],
    compiler_params=pltpu.CompilerParams(use_tc_tiling_on_sc=False),
)
def _gather_scatter(h_hbm, ee_hbm, srci_hbm, dsti_hbm, out_hbm,
                    srcall, dstall, hb, eb, sb, zero_v, acc,
                    sem_h, sem_e, sem_s):
    c = lax.axis_index("c")
    s = lax.axis_index("s")
    base0 = s * EPT
    base2 = s * (EPT // 2)

    pltpu.sync_copy(srci_hbm.at[s], srcall)
    pltpu.sync_copy(dsti_hbm.at[s], dstall)

    for b in range(2):
        pltpu.async_copy(ee_hbm.at[c, pl.ds(base2 + b * (C // 2), C // 2)],
                         eb[b], sem_e[b])
        pltpu.async_copy(h_hbm.at[c].at[srcall.at[b]], hb[b], sem_h[b])

    def zbody(i, carry):
        for j in range(DH // 16):
            zero_v[i, pl.ds(j * 16, 16)] = jnp.zeros((16,), jnp.float32)
        return carry

    lax.fori_loop(0, ZR, zbody, None)
    r0 = s * ROWS_PT
    for k in range(ROWS_PT // ZR):
        pltpu.sync_copy(zero_v, acc.at[pl.ds(r0 + k * ZR, ZR)])
    pltpu.sync_copy(zero_v.at[pl.ds(0, ROWS_PT % ZR)],
                    acc.at[pl.ds(r0 + (ROWS_PT // ZR) * ZR, ROWS_PT % ZR)])
    plsc.subcore_barrier()

    def outer(g, carry):
        for b in range(2):
            i = 2 * g + b
            pltpu.make_async_copy(
                ee_hbm.at[c, pl.ds(base2 + i * (C // 2), C // 2)],
                eb[b], sem_e[b]).wait()
            pltpu.make_async_copy(h_hbm.at[c].at[srcall.at[i]], hb[b],
                                  sem_h[b]).wait()

            @pl.when(g > 0)
            def _():
                pltpu.make_async_copy(sb[b], acc.at[dstall.at[i - 2]],
                                      sem_s[b]).wait()

            def relu_pair(q, inner):
                for par in range(2):
                    for j in range(DH // 16):
                        sl = pl.ds(j * 16, 16)
                        esl = pl.ds(par * DH + j * 16, 16)
                        sb[b][2 * q + par, sl] = jnp.maximum(
                            hb[b][2 * q + par, sl] + eb[b][q, esl], 0.0)
                return inner

            lax.fori_loop(0, C // 2, relu_pair, None)

            @pl.when(g < NOUTER - 1)
            def _():
                nxt = i + 2
                pltpu.async_copy(
                    ee_hbm.at[c, pl.ds(base2 + nxt * (C // 2), C // 2)],
                    eb[b], sem_e[b])
                pltpu.async_copy(h_hbm.at[c].at[srcall.at[nxt]], hb[b],
                                 sem_h[b])

            pltpu.async_copy(sb[b], acc.at[dstall.at[i]], sem_s[b],
                             add=True)
        return carry

    lax.fori_loop(0, NOUTER, outer, None)

    for b in range(2):
        pltpu.make_async_copy(sb[b], acc.at[dstall.at[NCHUNK - 2 + b]],
                              sem_s[b]).wait()

    plsc.subcore_barrier()
    pltpu.sync_copy(acc.at[pl.ds(r0, ROWS_PT)],
                    out_hbm.at[c, pl.ds(r0, ROWS_PT)])


def _enc_body(x_ref, w_ref, b_ref, o_ref, oh_ref):
    h = (
        jnp.dot(x_ref[...], w_ref[...], preferred_element_type=jnp.float32)
        + b_ref[...]
    )
    o_ref[...] = h
    oh_ref[0] = h[:, :DH]
    oh_ref[1] = h[:, DH:]


def _encoder(x, W0, b0):
    return pl.pallas_call(
        _enc_body,
        out_shape=[
            jax.ShapeDtypeStruct((N, D), jnp.float32),
            jax.ShapeDtypeStruct((NC, N, DH), jnp.float32),
        ],
    )(x, W0, b0.reshape(1, D))


_EB = 3200


def _ee_body(ea_ref, w_ref, b_ref, o0_ref, o1_ref, o2_ref):
    p = (
        jnp.dot(ea_ref[...], w_ref[...], preferred_element_type=jnp.float32)
        + b_ref[...]
    )
    for l, o_ref in enumerate((o0_ref, o1_ref, o2_ref)):
        for cc in range(NC):
            a = l * D + cc * DH
            o_ref[cc] = jnp.concatenate(
                [p[:, a:a + DH], p[:, L * D + a:L * D + a + DH]], axis=1)


def _ee_all(edge_attr, We, be):
    Wc = jnp.transpose(We, (1, 0, 2)).reshape(DE, L * D)
    z = jnp.zeros((DE, L * D), jnp.float32)
    Wc2 = jnp.concatenate(
        [jnp.concatenate([Wc, z], axis=1), jnp.concatenate([z, Wc], axis=1)],
        axis=0)
    bc2 = jnp.concatenate([be.reshape(1, L * D)] * 2, axis=1)
    ea2 = edge_attr.reshape(E // 2, 2 * DE)
    eb2 = _EB // 2
    return pl.pallas_call(
        _ee_body,
        grid=(E // _EB,),
        in_specs=[
            pl.BlockSpec((eb2, 2 * DE), lambda i: (i, 0)),
            pl.BlockSpec((2 * DE, 2 * L * D), lambda i: (0, 0)),
            pl.BlockSpec((1, 2 * L * D), lambda i: (0, 0)),
        ],
        out_specs=[pl.BlockSpec((NC, eb2, D), lambda i: (0, i, 0))] * 3,
        out_shape=[jax.ShapeDtypeStruct((NC, E // 2, D), jnp.float32)] * 3,
    )(ea2, Wc2, bc2)


def _dense_body(relu_out, h_ref, p_ref, eps_ref, w1_ref, b1_ref, g1_ref,
                bb1_ref, w2_ref, b2_ref, go_ref, bo_ref, o_ref, oh_ref):
    aggr = jnp.concatenate([p_ref[0, :N], p_ref[1, :N]], axis=1)
    u = (1.0 + eps_ref[0]) * h_ref[...] + aggr
    t = (
        jnp.dot(u, w1_ref[...], preferred_element_type=jnp.float32)
        + b1_ref[...]
    )
    m = jnp.mean(t, axis=0, keepdims=True)
    v = jnp.mean(jnp.square(t - m), axis=0, keepdims=True)
    t = (t - m) * lax.rsqrt(v + 1e-5) * g1_ref[...] + bb1_ref[...]
    t = jnp.maximum(t, 0.0)
    o = (
        jnp.dot(t, w2_ref[...], preferred_element_type=jnp.float32)
        + b2_ref[...]
    )
    m2 = jnp.mean(o, axis=0, keepdims=True)
    v2 = jnp.mean(jnp.square(o - m2), axis=0, keepdims=True)
    o = (o - m2) * lax.rsqrt(v2 + 1e-5) * go_ref[...] + bo_ref[...]
    if relu_out:
        o = jnp.maximum(o, 0.0)
    o_ref[...] = o
    oh_ref[0] = o[:, :DH]
    oh_ref[1] = o[:, DH:]


def _dense(relu_out, h, p, eps_l, W1l, b1l, g1l, bb1l, W2l, b2l, gOl, bOl):
    vspec = pl.BlockSpec(memory_space=pltpu.MemorySpace.VMEM)
    sspec = pl.BlockSpec(memory_space=pltpu.SMEM)
    return pl.pallas_call(
        functools.partial(_dense_body, relu_out),
        in_specs=[vspec, vspec, sspec] + [vspec] * 8,
        out_specs=[vspec, vspec],
        out_shape=[
            jax.ShapeDtypeStruct((N, D), jnp.float32),
            jax.ShapeDtypeStruct((NC, N, DH), jnp.float32),
        ],
    )(h, p, eps_l.reshape(1), W1l, b1l.reshape(1, 2 * D),
      g1l.reshape(1, 2 * D), bb1l.reshape(1, 2 * D), W2l,
      b2l.reshape(1, D), gOl.reshape(1, D), bOl.reshape(1, D))


def kernel(x, edge_index, edge_attr, batch, W0, b0, We, be, W1, b1, g1, bb1,
           W2, b2, gO, bO, eps):
    src = edge_index[0].reshape(NS, NCHUNK, C)
    dsti = edge_index[1].reshape(NS, NCHUNK, C)
    h, hh = _encoder(x, W0, b0)
    ee = _ee_all(edge_attr, We, be)
    for l in range(L):
        partials = _gather_scatter(hh, ee[l], src, dsti)
        h, hh = _dense(l < L - 1, h, partials, eps[l], W1[l], b1[l], g1[l],
                       bb1[l], W2[l], b2[l], gO[l], bO[l])
    return h
